# Initial kernel scaffold; baseline (speedup 1.0000x reference)
import jax
import jax.numpy as jnp
from jax.experimental import pallas as pl


def kernel(x_embed, prompt_key):
    raise NotImplementedError("write your pallas kernel here")


# speedup vs baseline: 1.0632x; 1.0632x over previous
import functools

import jax
import jax.numpy as jnp
from jax import lax
from jax.experimental import pallas as pl
from jax.experimental.pallas import tpu as pltpu

B = 256
T = 197
D = 768
P = 16384
K = 8

_B_BLK = 32
_P_BLK = 2048
_TK_BLK = 32


def _mean_norm_body(x_ref, o_ref):
    x = x_ref[...]
    m = jnp.sum(x, axis=1) * (1.0 / T)
    sq = jnp.sum(m * m, axis=-1, keepdims=True)
    o_ref[...] = m * lax.rsqrt(jnp.maximum(sq, 1e-12))


def _sim_body(q_ref, k_ref, o_ref):
    k = k_ref[...]
    q = q_ref[...]
    ksq = jnp.sum(k * k, axis=-1, keepdims=True)
    kn = k * lax.rsqrt(jnp.maximum(ksq, 1e-12))
    s = lax.dot_general(q.astype(jnp.bfloat16), kn.astype(jnp.bfloat16),
                        (((1,), (1,)), ((), ())),
                        preferred_element_type=jnp.float32)
    o_ref[...] = s


def _topk_body(s_ref, v_ref, i_ref):
    v = s_ref[...]
    col = lax.broadcasted_iota(jnp.int32, v.shape, 1)
    vals = []
    idxs = []
    for _ in range(K):
        m = jnp.max(v, axis=1, keepdims=True)
        hit = v == m
        idx = jnp.min(jnp.where(hit, col, jnp.int32(2**30)), axis=1,
                      keepdims=True)
        vals.append(m)
        idxs.append(idx)
        v = jnp.where(col == idx, -jnp.inf, v)
    v_ref[...] = jnp.concatenate(vals, axis=1)
    i_ref[...] = jnp.concatenate(idxs, axis=1)


@jax.jit
def kernel(x_embed, prompt_key):
    q_norm = pl.pallas_call(
        _mean_norm_body,
        grid=(B // _B_BLK,),
        in_specs=[pl.BlockSpec((_B_BLK, T, D), lambda i: (i, 0, 0))],
        out_specs=pl.BlockSpec((_B_BLK, D), lambda i: (i, 0)),
        out_shape=jax.ShapeDtypeStruct((B, D), jnp.float32),
    )(x_embed)

    sim = pl.pallas_call(
        _sim_body,
        grid=(P // _P_BLK,),
        in_specs=[
            pl.BlockSpec((B, D), lambda j: (0, 0)),
            pl.BlockSpec((_P_BLK, D), lambda j: (j, 0)),
        ],
        out_specs=pl.BlockSpec((B, _P_BLK), lambda j: (0, j)),
        out_shape=jax.ShapeDtypeStruct((B, P), jnp.float32),
    )(q_norm, prompt_key)

    top_v, top_i = pl.pallas_call(
        _topk_body,
        grid=(B // _TK_BLK,),
        in_specs=[pl.BlockSpec((_TK_BLK, P), lambda i: (i, 0))],
        out_specs=[
            pl.BlockSpec((_TK_BLK, K), lambda i: (i, 0)),
            pl.BlockSpec((_TK_BLK, K), lambda i: (i, 0)),
        ],
        out_shape=[
            jax.ShapeDtypeStruct((B, K), jnp.float32),
            jax.ShapeDtypeStruct((B, K), jnp.int32),
        ],
    )(sim)

    return sim, top_v, top_i


# speedup vs baseline: 1.5558x; 1.4633x over previous
import functools

import jax
import jax.numpy as jnp
from jax import lax
from jax.experimental import pallas as pl
from jax.experimental.pallas import tpu as pltpu

B = 256
T = 197
D = 768
P = 16384
K = 8

_P_BLK = 2048
_TK_BLK = 32


def _l2_normalize(x):
    sq = jnp.sum(x * x, axis=-1, keepdims=True)
    return x * lax.rsqrt(jnp.maximum(sq, 1e-12))


def _sim_body(q_ref, k_ref, o_ref):
    s = lax.dot_general(q_ref[...].astype(jnp.bfloat16),
                        k_ref[...].astype(jnp.bfloat16),
                        (((1,), (1,)), ((), ())),
                        preferred_element_type=jnp.float32)
    o_ref[...] = s


def _topk_body(s_ref, v_ref, i_ref):
    v = s_ref[...]
    col = lax.broadcasted_iota(jnp.int32, v.shape, 1)
    vals = []
    idxs = []
    for _ in range(K):
        m = jnp.max(v, axis=1, keepdims=True)
        hit = v == m
        idx = jnp.min(jnp.where(hit, col, jnp.int32(2**30)), axis=1,
                      keepdims=True)
        vals.append(m)
        idxs.append(idx)
        v = jnp.where(col == idx, -jnp.inf, v)
    v_ref[...] = jnp.concatenate(vals, axis=1)
    i_ref[...] = jnp.concatenate(idxs, axis=1)


@jax.jit
def kernel(x_embed, prompt_key):
    q_norm = _l2_normalize(jnp.mean(x_embed, axis=1))
    key_norm = _l2_normalize(prompt_key)

    sim = pl.pallas_call(
        _sim_body,
        grid=(P // _P_BLK,),
        in_specs=[
            pl.BlockSpec((B, D), lambda j: (0, 0)),
            pl.BlockSpec((_P_BLK, D), lambda j: (j, 0)),
        ],
        out_specs=pl.BlockSpec((B, _P_BLK), lambda j: (0, j)),
        out_shape=jax.ShapeDtypeStruct((B, P), jnp.float32),
    )(q_norm, key_norm)

    top_v, top_i = pl.pallas_call(
        _topk_body,
        grid=(B // _TK_BLK,),
        in_specs=[pl.BlockSpec((_TK_BLK, P), lambda i: (i, 0))],
        out_specs=[
            pl.BlockSpec((_TK_BLK, K), lambda i: (i, 0)),
            pl.BlockSpec((_TK_BLK, K), lambda i: (i, 0)),
        ],
        out_shape=[
            jax.ShapeDtypeStruct((B, K), jnp.float32),
            jax.ShapeDtypeStruct((B, K), jnp.int32),
        ],
    )(sim)

    return sim, top_v, top_i
